# baseline (speedup 1.0000x reference)
import functools

import jax
import jax.numpy as jnp
from jax import lax
from jax.experimental import pallas as pl
from jax.experimental.pallas import tpu as pltpu
from jax.experimental.pallas import tpu_sc as plsc

_N = 10000
_NPAD = 10240
_E = 320000
_NC = 2
_NS = 16
_CHUNK = 128
_CPW = 160
_EPW = _CPW * _CHUNK
_EPAD = _EPW * _NS
_ROWS_PS = _NPAD // _NS
_NBUF = 8
_NHBM = 2
_BLK = 1280


def _sc_mesh():
    return plsc.VectorSubcoreMesh(core_axis_name="c", subcore_axis_name="s")


@functools.lru_cache(maxsize=None)
def _make_scatter(F):
    HF = F // 2

    def body(gh_hbm, src_hbm, dst_hbm, out_hbm,
             src_v, dst_v, bufs, semg, sems, acc, gsp):
        cid = lax.axis_index("c")
        sid = lax.axis_index("s")
        slab = pl.ds(sid * _ROWS_PS, _ROWS_PS)
        cols = pl.ds(cid * HF, HF)

        pltpu.sync_copy(gh_hbm.at[cid, slab], gsp.at[slab])
        pltpu.sync_copy(gh_hbm.at[cid, slab], acc.at[slab])
        pltpu.sync_copy(src_hbm.at[sid], src_v)
        pltpu.sync_copy(dst_hbm.at[sid], dst_v)
        plsc.subcore_barrier()

        def src_ref(b):
            return gh_hbm.at[cid] if b < _NHBM else gsp

        for b in range(_NBUF):
            pltpu.async_copy(src_ref(b).at[src_v.at[b]], bufs[b], semg[b])

        def step(p, carry):
            base = p * _NBUF
            for b in range(_NBUF):
                j = base + b
                pltpu.make_async_copy(src_ref(b).at[src_v.at[j]], bufs[b],
                                      semg[b]).wait()
                pltpu.async_copy(bufs[b], acc.at[dst_v.at[j]], sems[b],
                                 add=True)
            for b in range(_NBUF):
                j2 = base + _NBUF + b

                @pl.when(j2 < _CPW)
                def _():
                    pltpu.make_async_copy(bufs[b], acc.at[dst_v.at[base + b]],
                                          sems[b]).wait()
                    pltpu.async_copy(src_ref(b).at[src_v.at[j2]], bufs[b],
                                     semg[b])

            return carry

        lax.fori_loop(0, _CPW // _NBUF, step, 0)
        for b in range(_NBUF):
            j = _CPW - _NBUF + b
            pltpu.make_async_copy(bufs[b], acc.at[dst_v.at[j]], sems[b]).wait()
        plsc.subcore_barrier()
        pltpu.sync_copy(acc.at[slab], out_hbm.at[slab, cols])

    return pl.kernel(
        body,
        out_type=jax.ShapeDtypeStruct((_NPAD, F), jnp.float32),
        mesh=_sc_mesh(),
        compiler_params=pltpu.CompilerParams(use_tc_tiling_on_sc=False),
        scratch_types=[
            pltpu.VMEM((_CPW, _CHUNK), jnp.int32),
            pltpu.VMEM((_CPW, _CHUNK), jnp.int32),
            [pltpu.VMEM((_CHUNK, HF), jnp.float32) for _ in range(_NBUF)],
            [pltpu.SemaphoreType.DMA for _ in range(_NBUF)],
            [pltpu.SemaphoreType.DMA for _ in range(_NBUF)],
            pltpu.VMEM_SHARED((_NPAD, HF), jnp.float32),
            pltpu.VMEM_SHARED((_NPAD, HF), jnp.float32),
        ],
    )


_CPWD = 80


def _make_deg():
    def body(ones_hbm, z_hbm, dst_hbm, out_hbm, dst_v, ones_v, sem, acc):
        cid = lax.axis_index("c")
        sid = lax.axis_index("s")
        wid = cid * _NS + sid
        slab = pl.ds(sid * _ROWS_PS, _ROWS_PS)
        pltpu.sync_copy(z_hbm.at[slab], acc.at[slab])
        pltpu.sync_copy(dst_hbm.at[wid], dst_v)
        pltpu.sync_copy(ones_hbm, ones_v)
        plsc.subcore_barrier()

        def fire(j, carry):
            pltpu.async_copy(ones_v, acc.at[dst_v.at[j]], sem, add=True)
            return carry

        lax.fori_loop(0, _CPWD, fire, 0)

        def drain(j, carry):
            pltpu.make_async_copy(ones_v, acc.at[dst_v.at[j]], sem).wait()
            return carry

        lax.fori_loop(0, _CPWD, drain, 0)
        plsc.subcore_barrier()
        pltpu.sync_copy(acc.at[slab], out_hbm.at[cid, slab])

    return pl.kernel(
        body,
        out_type=jax.ShapeDtypeStruct((_NC, _NPAD, 16), jnp.float32),
        mesh=_sc_mesh(),
        compiler_params=pltpu.CompilerParams(use_tc_tiling_on_sc=False),
        scratch_types=[
            pltpu.VMEM((_CPWD, _CHUNK), jnp.int32),
            pltpu.VMEM((_CHUNK, 16), jnp.float32),
            pltpu.SemaphoreType.DMA,
            pltpu.VMEM_SHARED((_NPAD, 16), jnp.float32),
        ],
    )


def _dinv_from(degp_ref):
    deg = degp_ref[0, :, 0:1] + degp_ref[1, :, 0:1] + 1.0
    return lax.rsqrt(deg)


def _first_body(x_ref, w_ref, degp_ref, g_ref):
    dinv = _dinv_from(degp_ref)
    g_ref[...] = dinv * jnp.dot(x_ref[...], w_ref[...],
                                preferred_element_type=jnp.float32)


def _mid2_body(sa_ref, sb_ref, degp_ref, b_ref, wa_ref, wb_ref, g_ref):
    dinv = _dinv_from(degp_ref)
    hw = b_ref.shape[-1] // 2
    aa = jnp.maximum(dinv * sa_ref[...] + b_ref[0:1, :hw], 0.0)
    ab = jnp.maximum(dinv * sb_ref[...] + b_ref[0:1, hw:], 0.0)
    g_ref[...] = dinv * (
        jnp.dot(aa, wa_ref[...], preferred_element_type=jnp.float32)
        + jnp.dot(ab, wb_ref[...], preferred_element_type=jnp.float32))


def _mid_body(s_ref, degp_ref, b_ref, w_ref, g_ref):
    dinv = _dinv_from(degp_ref)
    a = dinv * s_ref[...] + b_ref[0:1, :]
    a = jnp.maximum(a, 0.0)
    g_ref[...] = dinv * jnp.dot(a, w_ref[...], preferred_element_type=jnp.float32)


def _last_body(s_ref, degp_ref, b_ref, o_ref):
    dinv = _dinv_from(degp_ref)
    o_ref[...] = dinv * s_ref[...] + b_ref[0:1, :]


def _row_spec(F):
    return pl.BlockSpec((_BLK, F), lambda i: (i, 0))


def _partials_spec(F):
    return pl.BlockSpec((_NC, _BLK, F), lambda i: (0, i, 0))


def _full_spec(shape):
    nd = len(shape)
    return pl.BlockSpec(shape, lambda i: (0,) * nd)


def _tc_first(x_pad, W, degp):
    return pl.pallas_call(
        _first_body,
        grid=(_NPAD // _BLK,),
        in_specs=[_row_spec(128), _full_spec(W.shape), _partials_spec(16)],
        out_specs=_row_spec(W.shape[1]),
        out_shape=jax.ShapeDtypeStruct((_NPAD, W.shape[1]), jnp.float32),
    )(x_pad, W, degp)


def _tc_mid(s, degp, b8, W):
    F1, F2 = W.shape
    return pl.pallas_call(
        _mid_body,
        grid=(_NPAD // _BLK,),
        in_specs=[_row_spec(F1), _partials_spec(16), _full_spec(b8.shape),
                  _full_spec(W.shape)],
        out_specs=_row_spec(F2),
        out_shape=jax.ShapeDtypeStruct((_NPAD, F2), jnp.float32),
    )(s, degp, b8, W)


def _tc_mid2(sa, sb, degp, b8, Wa, Wb):
    F2 = Wa.shape[1]
    half = sa.shape[-1]
    return pl.pallas_call(
        _mid2_body,
        grid=(_NPAD // _BLK,),
        in_specs=[_row_spec(half), _row_spec(half), _partials_spec(16),
                  _full_spec(b8.shape), _full_spec(Wa.shape), _full_spec(Wb.shape)],
        out_specs=_row_spec(F2),
        out_shape=jax.ShapeDtypeStruct((_NPAD, F2), jnp.float32),
    )(sa, sb, degp, b8, Wa, Wb)


def _tc_last(s, degp, b8):
    F = s.shape[-1]
    return pl.pallas_call(
        _last_body,
        grid=(_NPAD // _BLK,),
        in_specs=[_row_spec(F), _partials_spec(16), _full_spec(b8.shape)],
        out_specs=_row_spec(F),
        out_shape=jax.ShapeDtypeStruct((_NPAD, F), jnp.float32),
    )(s, degp, b8)


def kernel(x, adj_t, W1, b1, W2, b2, W3, b3):
    src = adj_t[0].astype(jnp.int32)
    dst = adj_t[1].astype(jnp.int32)
    pad = _EPAD - _E
    src_flat = jnp.concatenate([src, jnp.zeros((pad,), jnp.int32)])
    dst_flat = jnp.concatenate([dst, jnp.full((pad,), _N, jnp.int32)])
    srcp = src_flat.reshape(_NS, _CPW, _CHUNK)
    dstp = dst_flat.reshape(_NS, _CPW, _CHUNK)
    dstp_deg = dst_flat.reshape(_NC * _NS, _CPWD, _CHUNK)
    x_pad = jnp.pad(x, ((0, _NPAD - _N), (0, 0)))
    ones16 = jnp.ones((_CHUNK, 16), jnp.float32)

    degp = _make_deg()(ones16, jnp.zeros((_NPAD, 16), jnp.float32), dstp_deg)

    g1 = _tc_first(x_pad, W1, degp)
    scat64 = _make_scatter(64)

    def halves(g):
        hf = g.shape[1] // 2
        return jnp.stack([g[:, :hf], g[:, hf:]])

    s1a = scat64(halves(g1[:, :64]), srcp, dstp)
    s1b = scat64(halves(g1[:, 64:]), srcp, dstp)

    b1_8 = jnp.tile(b1[None, :], (8, 1))
    g2 = _tc_mid2(s1a, s1b, degp, b1_8, W2[:64], W2[64:])
    s2 = scat64(halves(g2), srcp, dstp)

    W3p = jnp.pad(W3, ((0, 0), (0, 24)))
    b2_8 = jnp.tile(b2[None, :], (8, 1))
    g3 = _tc_mid(s2, degp, b2_8, W3p)
    s3 = scat64(halves(g3), srcp, dstp)

    b3_8 = jnp.tile(jnp.pad(b3, (0, 24))[None, :], (8, 1))
    out = _tc_last(s3, degp, b3_8)
    return out[:_N, :40]


# speedup vs baseline: 1.0097x; 1.0097x over previous
import functools

import jax
import jax.numpy as jnp
from jax import lax
from jax.experimental import pallas as pl
from jax.experimental.pallas import tpu as pltpu
from jax.experimental.pallas import tpu_sc as plsc

_N = 10000
_NPAD = 10240
_E = 320000
_NC = 2
_NS = 16
_CHUNK = 128
_CPW = 160
_EPW = _CPW * _CHUNK
_EPAD = _EPW * _NS
_ROWS_PS = _NPAD // _NS
_NBUF = 8
_NHBM = 0
_BLK = 1280


def _sc_mesh():
    return plsc.VectorSubcoreMesh(core_axis_name="c", subcore_axis_name="s")


@functools.lru_cache(maxsize=None)
def _make_scatter(F):
    HF = F // 2

    def body(gh_hbm, src_hbm, dst_hbm, out_hbm,
             src_v, dst_v, bufs, semg, sems, acc, gsp):
        cid = lax.axis_index("c")
        sid = lax.axis_index("s")
        slab = pl.ds(sid * _ROWS_PS, _ROWS_PS)
        cols = pl.ds(cid * HF, HF)

        pltpu.sync_copy(gh_hbm.at[cid, slab], gsp.at[slab])
        pltpu.sync_copy(gh_hbm.at[cid, slab], acc.at[slab])
        pltpu.sync_copy(src_hbm.at[sid], src_v)
        pltpu.sync_copy(dst_hbm.at[sid], dst_v)
        plsc.subcore_barrier()

        def src_ref(b):
            return gh_hbm.at[cid] if b < _NHBM else gsp

        for b in range(_NBUF):
            pltpu.async_copy(src_ref(b).at[src_v.at[b]], bufs[b], semg[b])

        def step(p, carry):
            base = p * _NBUF
            for b in range(_NBUF):
                j = base + b
                pltpu.make_async_copy(src_ref(b).at[src_v.at[j]], bufs[b],
                                      semg[b]).wait()
                pltpu.async_copy(bufs[b], acc.at[dst_v.at[j]], sems[b],
                                 add=True)
            for b in range(_NBUF):
                j2 = base + _NBUF + b

                @pl.when(j2 < _CPW)
                def _():
                    pltpu.make_async_copy(bufs[b], acc.at[dst_v.at[base + b]],
                                          sems[b]).wait()
                    pltpu.async_copy(src_ref(b).at[src_v.at[j2]], bufs[b],
                                     semg[b])

            return carry

        lax.fori_loop(0, _CPW // _NBUF, step, 0)
        for b in range(_NBUF):
            j = _CPW - _NBUF + b
            pltpu.make_async_copy(bufs[b], acc.at[dst_v.at[j]], sems[b]).wait()
        plsc.subcore_barrier()
        pltpu.sync_copy(acc.at[slab], out_hbm.at[slab, cols])

    return pl.kernel(
        body,
        out_type=jax.ShapeDtypeStruct((_NPAD, F), jnp.float32),
        mesh=_sc_mesh(),
        compiler_params=pltpu.CompilerParams(use_tc_tiling_on_sc=False),
        scratch_types=[
            pltpu.VMEM((_CPW, _CHUNK), jnp.int32),
            pltpu.VMEM((_CPW, _CHUNK), jnp.int32),
            [pltpu.VMEM((_CHUNK, HF), jnp.float32) for _ in range(_NBUF)],
            [pltpu.SemaphoreType.DMA for _ in range(_NBUF)],
            [pltpu.SemaphoreType.DMA for _ in range(_NBUF)],
            pltpu.VMEM_SHARED((_NPAD, HF), jnp.float32),
            pltpu.VMEM_SHARED((_NPAD, HF), jnp.float32),
        ],
    )


_CPWD = 80


def _make_deg():
    def body(ones_hbm, z_hbm, dst_hbm, out_hbm, dst_v, ones_v, sem, acc):
        cid = lax.axis_index("c")
        sid = lax.axis_index("s")
        wid = cid * _NS + sid
        slab = pl.ds(sid * _ROWS_PS, _ROWS_PS)
        pltpu.sync_copy(z_hbm.at[slab], acc.at[slab])
        pltpu.sync_copy(dst_hbm.at[wid], dst_v)
        pltpu.sync_copy(ones_hbm, ones_v)
        plsc.subcore_barrier()

        def fire(j, carry):
            pltpu.async_copy(ones_v, acc.at[dst_v.at[j]], sem, add=True)
            return carry

        lax.fori_loop(0, _CPWD, fire, 0)

        def drain(j, carry):
            pltpu.make_async_copy(ones_v, acc.at[dst_v.at[j]], sem).wait()
            return carry

        lax.fori_loop(0, _CPWD, drain, 0)
        plsc.subcore_barrier()
        pltpu.sync_copy(acc.at[slab], out_hbm.at[cid, slab])

    return pl.kernel(
        body,
        out_type=jax.ShapeDtypeStruct((_NC, _NPAD, 16), jnp.float32),
        mesh=_sc_mesh(),
        compiler_params=pltpu.CompilerParams(use_tc_tiling_on_sc=False),
        scratch_types=[
            pltpu.VMEM((_CPWD, _CHUNK), jnp.int32),
            pltpu.VMEM((_CHUNK, 16), jnp.float32),
            pltpu.SemaphoreType.DMA,
            pltpu.VMEM_SHARED((_NPAD, 16), jnp.float32),
        ],
    )


def _dinv_from(degp_ref):
    deg = degp_ref[0, :, 0:1] + degp_ref[1, :, 0:1] + 1.0
    return lax.rsqrt(deg)


def _first_body(x_ref, w_ref, degp_ref, g_ref):
    dinv = _dinv_from(degp_ref)
    g_ref[...] = dinv * jnp.dot(x_ref[...], w_ref[...],
                                preferred_element_type=jnp.float32)


def _mid2_body(sa_ref, sb_ref, degp_ref, b_ref, wa_ref, wb_ref, g_ref):
    dinv = _dinv_from(degp_ref)
    hw = b_ref.shape[-1] // 2
    aa = jnp.maximum(dinv * sa_ref[...] + b_ref[0:1, :hw], 0.0)
    ab = jnp.maximum(dinv * sb_ref[...] + b_ref[0:1, hw:], 0.0)
    g_ref[...] = dinv * (
        jnp.dot(aa, wa_ref[...], preferred_element_type=jnp.float32)
        + jnp.dot(ab, wb_ref[...], preferred_element_type=jnp.float32))


def _mid_body(s_ref, degp_ref, b_ref, w_ref, g_ref):
    dinv = _dinv_from(degp_ref)
    a = dinv * s_ref[...] + b_ref[0:1, :]
    a = jnp.maximum(a, 0.0)
    g_ref[...] = dinv * jnp.dot(a, w_ref[...], preferred_element_type=jnp.float32)


def _last_body(s_ref, degp_ref, b_ref, o_ref):
    dinv = _dinv_from(degp_ref)
    o_ref[...] = dinv * s_ref[...] + b_ref[0:1, :]


def _row_spec(F):
    return pl.BlockSpec((_BLK, F), lambda i: (i, 0))


def _partials_spec(F):
    return pl.BlockSpec((_NC, _BLK, F), lambda i: (0, i, 0))


def _full_spec(shape):
    nd = len(shape)
    return pl.BlockSpec(shape, lambda i: (0,) * nd)


def _tc_first(x_pad, W, degp):
    return pl.pallas_call(
        _first_body,
        grid=(_NPAD // _BLK,),
        in_specs=[_row_spec(128), _full_spec(W.shape), _partials_spec(16)],
        out_specs=_row_spec(W.shape[1]),
        out_shape=jax.ShapeDtypeStruct((_NPAD, W.shape[1]), jnp.float32),
    )(x_pad, W, degp)


def _tc_mid(s, degp, b8, W):
    F1, F2 = W.shape
    return pl.pallas_call(
        _mid_body,
        grid=(_NPAD // _BLK,),
        in_specs=[_row_spec(F1), _partials_spec(16), _full_spec(b8.shape),
                  _full_spec(W.shape)],
        out_specs=_row_spec(F2),
        out_shape=jax.ShapeDtypeStruct((_NPAD, F2), jnp.float32),
    )(s, degp, b8, W)


def _tc_mid2(sa, sb, degp, b8, Wa, Wb):
    F2 = Wa.shape[1]
    half = sa.shape[-1]
    return pl.pallas_call(
        _mid2_body,
        grid=(_NPAD // _BLK,),
        in_specs=[_row_spec(half), _row_spec(half), _partials_spec(16),
                  _full_spec(b8.shape), _full_spec(Wa.shape), _full_spec(Wb.shape)],
        out_specs=_row_spec(F2),
        out_shape=jax.ShapeDtypeStruct((_NPAD, F2), jnp.float32),
    )(sa, sb, degp, b8, Wa, Wb)


def _tc_last(s, degp, b8):
    F = s.shape[-1]
    return pl.pallas_call(
        _last_body,
        grid=(_NPAD // _BLK,),
        in_specs=[_row_spec(F), _partials_spec(16), _full_spec(b8.shape)],
        out_specs=_row_spec(F),
        out_shape=jax.ShapeDtypeStruct((_NPAD, F), jnp.float32),
    )(s, degp, b8)


def kernel(x, adj_t, W1, b1, W2, b2, W3, b3):
    src = adj_t[0].astype(jnp.int32)
    dst = adj_t[1].astype(jnp.int32)
    pad = _EPAD - _E
    src_flat = jnp.concatenate([src, jnp.zeros((pad,), jnp.int32)])
    dst_flat = jnp.concatenate([dst, jnp.full((pad,), _N, jnp.int32)])
    srcp = src_flat.reshape(_NS, _CPW, _CHUNK)
    dstp = dst_flat.reshape(_NS, _CPW, _CHUNK)
    dstp_deg = dst_flat.reshape(_NC * _NS, _CPWD, _CHUNK)
    x_pad = jnp.pad(x, ((0, _NPAD - _N), (0, 0)))
    ones16 = jnp.ones((_CHUNK, 16), jnp.float32)

    degp = _make_deg()(ones16, jnp.zeros((_NPAD, 16), jnp.float32), dstp_deg)

    g1 = _tc_first(x_pad, W1, degp)
    scat64 = _make_scatter(64)

    def halves(g):
        hf = g.shape[1] // 2
        return jnp.stack([g[:, :hf], g[:, hf:]])

    s1a = scat64(halves(g1[:, :64]), srcp, dstp)
    s1b = scat64(halves(g1[:, 64:]), srcp, dstp)

    b1_8 = jnp.tile(b1[None, :], (8, 1))
    g2 = _tc_mid2(s1a, s1b, degp, b1_8, W2[:64], W2[64:])
    s2 = scat64(halves(g2), srcp, dstp)

    W3p = jnp.pad(W3, ((0, 0), (0, 24)))
    b2_8 = jnp.tile(b2[None, :], (8, 1))
    g3 = _tc_mid(s2, degp, b2_8, W3p)
    s3 = scat64(halves(g3), srcp, dstp)

    b3_8 = jnp.tile(jnp.pad(b3, (0, 24))[None, :], (8, 1))
    out = _tc_last(s3, degp, b3_8)
    return out[:_N, :40]


# speedup vs baseline: 1.0586x; 1.0485x over previous
import functools

import jax
import jax.numpy as jnp
from jax import lax
from jax.experimental import pallas as pl
from jax.experimental.pallas import tpu as pltpu
from jax.experimental.pallas import tpu_sc as plsc

_N = 10000
_NPAD = 10240
_E = 320000
_NC = 2
_NS = 16
_CHUNK = 128
_CPW = 160
_EPW = _CPW * _CHUNK
_EPAD = _EPW * _NS
_ROWS_PS = _NPAD // _NS
_NBUF = 8
_BLK = 1280


def _sc_mesh():
    return plsc.VectorSubcoreMesh(core_axis_name="c", subcore_axis_name="s")


@functools.lru_cache(maxsize=None)
def _make_scatter(F):
    HF = F // 2

    def body(g_hbm, src_hbm, dst_hbm, out_hbm,
             src_v, dst_v, bufs, semg, sems, acc, gsp):
        cid = lax.axis_index("c")
        sid = lax.axis_index("s")
        slab = pl.ds(sid * _ROWS_PS, _ROWS_PS)
        cols = pl.ds(cid * HF, HF)

        pltpu.sync_copy(g_hbm.at[slab, cols], gsp.at[slab])
        pltpu.sync_copy(g_hbm.at[slab, cols], acc.at[slab])
        pltpu.sync_copy(src_hbm.at[sid], src_v)
        pltpu.sync_copy(dst_hbm.at[sid], dst_v)
        plsc.subcore_barrier()

        for b in range(_NBUF):
            pltpu.async_copy(gsp.at[src_v.at[b]], bufs[b], semg[b])

        def step(p, carry):
            base = p * _NBUF
            for b in range(_NBUF):
                j = base + b
                pltpu.make_async_copy(gsp.at[src_v.at[j]], bufs[b],
                                      semg[b]).wait()
                pltpu.async_copy(bufs[b], acc.at[dst_v.at[j]], sems[b],
                                 add=True)
            for b in range(_NBUF):
                j2 = base + _NBUF + b

                @pl.when(j2 < _CPW)
                def _():
                    pltpu.make_async_copy(bufs[b], acc.at[dst_v.at[base + b]],
                                          sems[b]).wait()
                    pltpu.async_copy(gsp.at[src_v.at[j2]], bufs[b], semg[b])

            return carry

        lax.fori_loop(0, _CPW // _NBUF, step, 0)
        for b in range(_NBUF):
            j = _CPW - _NBUF + b
            pltpu.make_async_copy(bufs[b], acc.at[dst_v.at[j]], sems[b]).wait()
        plsc.subcore_barrier()
        pltpu.sync_copy(acc.at[slab], out_hbm.at[slab, cols])

    return pl.kernel(
        body,
        out_type=jax.ShapeDtypeStruct((_NPAD, F), jnp.float32),
        mesh=_sc_mesh(),
        compiler_params=pltpu.CompilerParams(use_tc_tiling_on_sc=False),
        scratch_types=[
            pltpu.VMEM((_CPW, _CHUNK), jnp.int32),
            pltpu.VMEM((_CPW, _CHUNK), jnp.int32),
            [pltpu.VMEM((_CHUNK, HF), jnp.float32) for _ in range(_NBUF)],
            [pltpu.SemaphoreType.DMA for _ in range(_NBUF)],
            [pltpu.SemaphoreType.DMA for _ in range(_NBUF)],
            pltpu.VMEM_SHARED((_NPAD, HF), jnp.float32),
            pltpu.VMEM_SHARED((_NPAD, HF), jnp.float32),
        ],
    )


_CPWD = 80


def _make_deg():
    def body(ones_hbm, z_hbm, dst_hbm, out_hbm, dst_v, ones_v, sem, acc):
        cid = lax.axis_index("c")
        sid = lax.axis_index("s")
        wid = cid * _NS + sid
        slab = pl.ds(sid * _ROWS_PS, _ROWS_PS)
        pltpu.sync_copy(z_hbm.at[slab], acc.at[slab])
        pltpu.sync_copy(dst_hbm.at[wid], dst_v)
        pltpu.sync_copy(ones_hbm, ones_v)
        plsc.subcore_barrier()

        def fire(j, carry):
            pltpu.async_copy(ones_v, acc.at[dst_v.at[j]], sem, add=True)
            return carry

        lax.fori_loop(0, _CPWD, fire, 0)

        def drain(j, carry):
            pltpu.make_async_copy(ones_v, acc.at[dst_v.at[j]], sem).wait()
            return carry

        lax.fori_loop(0, _CPWD, drain, 0)
        plsc.subcore_barrier()
        pltpu.sync_copy(acc.at[slab], out_hbm.at[cid, slab])

    return pl.kernel(
        body,
        out_type=jax.ShapeDtypeStruct((_NC, _NPAD, 16), jnp.float32),
        mesh=_sc_mesh(),
        compiler_params=pltpu.CompilerParams(use_tc_tiling_on_sc=False),
        scratch_types=[
            pltpu.VMEM((_CPWD, _CHUNK), jnp.int32),
            pltpu.VMEM((_CHUNK, 16), jnp.float32),
            pltpu.SemaphoreType.DMA,
            pltpu.VMEM_SHARED((_NPAD, 16), jnp.float32),
        ],
    )


def _dinv_from(degp_ref):
    deg = degp_ref[0, :, 0:1] + degp_ref[1, :, 0:1] + 1.0
    return lax.rsqrt(deg)


def _first_body(x_ref, w_ref, degp_ref, g_ref):
    dinv = _dinv_from(degp_ref)
    g_ref[...] = dinv * jnp.dot(x_ref[...], w_ref[...],
                                preferred_element_type=jnp.float32)


def _mid2_body(sa_ref, sb_ref, degp_ref, b_ref, wa_ref, wb_ref, g_ref):
    dinv = _dinv_from(degp_ref)
    hw = b_ref.shape[-1] // 2
    aa = jnp.maximum(dinv * sa_ref[...] + b_ref[0:1, :hw], 0.0)
    ab = jnp.maximum(dinv * sb_ref[...] + b_ref[0:1, hw:], 0.0)
    g_ref[...] = dinv * (
        jnp.dot(aa, wa_ref[...], preferred_element_type=jnp.float32)
        + jnp.dot(ab, wb_ref[...], preferred_element_type=jnp.float32))


def _mid_body(s_ref, degp_ref, b_ref, w_ref, g_ref):
    dinv = _dinv_from(degp_ref)
    a = dinv * s_ref[...] + b_ref[0:1, :]
    a = jnp.maximum(a, 0.0)
    g_ref[...] = dinv * jnp.dot(a, w_ref[...], preferred_element_type=jnp.float32)


def _last_body(s_ref, degp_ref, b_ref, o_ref):
    dinv = _dinv_from(degp_ref)
    o_ref[...] = dinv * s_ref[...] + b_ref[0:1, :]


def _row_spec(F):
    return pl.BlockSpec((_BLK, F), lambda i: (i, 0))


def _partials_spec(F):
    return pl.BlockSpec((_NC, _BLK, F), lambda i: (0, i, 0))


def _full_spec(shape):
    nd = len(shape)
    return pl.BlockSpec(shape, lambda i: (0,) * nd)


def _tc_first(x_pad, W, degp):
    return pl.pallas_call(
        _first_body,
        grid=(_NPAD // _BLK,),
        in_specs=[_row_spec(128), _full_spec(W.shape), _partials_spec(16)],
        out_specs=_row_spec(W.shape[1]),
        out_shape=jax.ShapeDtypeStruct((_NPAD, W.shape[1]), jnp.float32),
    )(x_pad, W, degp)


def _tc_mid(s, degp, b8, W):
    F1, F2 = W.shape
    return pl.pallas_call(
        _mid_body,
        grid=(_NPAD // _BLK,),
        in_specs=[_row_spec(F1), _partials_spec(16), _full_spec(b8.shape),
                  _full_spec(W.shape)],
        out_specs=_row_spec(F2),
        out_shape=jax.ShapeDtypeStruct((_NPAD, F2), jnp.float32),
    )(s, degp, b8, W)


def _tc_mid2(sa, sb, degp, b8, Wa, Wb):
    F2 = Wa.shape[1]
    half = sa.shape[-1]
    return pl.pallas_call(
        _mid2_body,
        grid=(_NPAD // _BLK,),
        in_specs=[_row_spec(half), _row_spec(half), _partials_spec(16),
                  _full_spec(b8.shape), _full_spec(Wa.shape), _full_spec(Wb.shape)],
        out_specs=_row_spec(F2),
        out_shape=jax.ShapeDtypeStruct((_NPAD, F2), jnp.float32),
    )(sa, sb, degp, b8, Wa, Wb)


def _tc_last(s, degp, b8):
    F = s.shape[-1]
    return pl.pallas_call(
        _last_body,
        grid=(_NPAD // _BLK,),
        in_specs=[_row_spec(F), _partials_spec(16), _full_spec(b8.shape)],
        out_specs=_row_spec(F),
        out_shape=jax.ShapeDtypeStruct((_NPAD, F), jnp.float32),
    )(s, degp, b8)


def kernel(x, adj_t, W1, b1, W2, b2, W3, b3):
    src = adj_t[0].astype(jnp.int32)
    dst = adj_t[1].astype(jnp.int32)
    pad = _EPAD - _E
    src_flat = jnp.concatenate([src, jnp.zeros((pad,), jnp.int32)])
    dst_flat = jnp.concatenate([dst, jnp.full((pad,), _N, jnp.int32)])
    srcp = src_flat.reshape(_NS, _CPW, _CHUNK)
    dstp = dst_flat.reshape(_NS, _CPW, _CHUNK)
    dstp_deg = dst_flat.reshape(_NC * _NS, _CPWD, _CHUNK)
    x_pad = jnp.pad(x, ((0, _NPAD - _N), (0, 0)))
    ones16 = jnp.ones((_CHUNK, 16), jnp.float32)

    degp = _make_deg()(ones16, jnp.zeros((_NPAD, 16), jnp.float32), dstp_deg)

    g1 = _tc_first(x_pad, W1, degp)
    scat64 = _make_scatter(64)
    s1a = scat64(g1[:, :64], srcp, dstp)
    s1b = scat64(g1[:, 64:], srcp, dstp)

    b1_8 = jnp.tile(b1[None, :], (8, 1))
    g2 = _tc_mid2(s1a, s1b, degp, b1_8, W2[:64], W2[64:])
    s2 = scat64(g2, srcp, dstp)

    W3p = jnp.pad(W3, ((0, 0), (0, 24)))
    b2_8 = jnp.tile(b2[None, :], (8, 1))
    g3 = _tc_mid(s2, degp, b2_8, W3p)
    s3 = scat64(g3, srcp, dstp)

    b3_8 = jnp.tile(jnp.pad(b3, (0, 24))[None, :], (8, 1))
    out = _tc_last(s3, degp, b3_8)
    return out[:_N, :40]


# speedup vs baseline: 1.0831x; 1.0232x over previous
import functools

import jax
import jax.numpy as jnp
from jax import lax
from jax.experimental import pallas as pl
from jax.experimental.pallas import tpu as pltpu
from jax.experimental.pallas import tpu_sc as plsc

_N = 10000
_NPAD = 10240
_E = 320000
_NC = 2
_NS = 16
_CHUNK = 128
_CPW = 160
_EPW = _CPW * _CHUNK
_EPAD = _EPW * _NS
_ROWS_PS = _NPAD // _NS
_NBUF = 8
_BLK = 1280


def _sc_mesh():
    return plsc.VectorSubcoreMesh(core_axis_name="c", subcore_axis_name="s")


@functools.lru_cache(maxsize=None)
def _make_scatter(F):
    HF = F // 2

    def body(g_hbm, src_hbm, dst_hbm, out_hbm,
             src_v, dst_v, bufs, semg, sems, acc, gsp):
        cid = lax.axis_index("c")
        sid = lax.axis_index("s")
        slab = pl.ds(sid * _ROWS_PS, _ROWS_PS)
        cols = pl.ds(cid * HF, HF)

        pltpu.async_copy(g_hbm.at[slab, cols], gsp.at[slab], semg[0])
        pltpu.async_copy(g_hbm.at[slab, cols], acc.at[slab], semg[1])
        pltpu.async_copy(src_hbm.at[sid], src_v, semg[2])
        pltpu.async_copy(dst_hbm.at[sid], dst_v, semg[3])
        pltpu.make_async_copy(g_hbm.at[slab, cols], gsp.at[slab], semg[0]).wait()
        pltpu.make_async_copy(g_hbm.at[slab, cols], acc.at[slab], semg[1]).wait()
        pltpu.make_async_copy(src_hbm.at[sid], src_v, semg[2]).wait()
        pltpu.make_async_copy(dst_hbm.at[sid], dst_v, semg[3]).wait()
        plsc.subcore_barrier()

        for b in range(_NBUF):
            pltpu.async_copy(gsp.at[src_v.at[b]], bufs[b], semg[b])

        def step(p, carry):
            base = p * _NBUF
            for b in range(_NBUF):
                j = base + b
                pltpu.make_async_copy(gsp.at[src_v.at[j]], bufs[b],
                                      semg[b]).wait()
                pltpu.async_copy(bufs[b], acc.at[dst_v.at[j]], sems[b],
                                 add=True)
            for b in range(_NBUF):
                j2 = base + _NBUF + b

                @pl.when(j2 < _CPW)
                def _():
                    pltpu.make_async_copy(bufs[b], acc.at[dst_v.at[base + b]],
                                          sems[b]).wait()
                    pltpu.async_copy(gsp.at[src_v.at[j2]], bufs[b], semg[b])

            return carry

        lax.fori_loop(0, _CPW // _NBUF, step, 0)
        for b in range(_NBUF):
            j = _CPW - _NBUF + b
            pltpu.make_async_copy(bufs[b], acc.at[dst_v.at[j]], sems[b]).wait()
        plsc.subcore_barrier()
        pltpu.sync_copy(acc.at[slab], out_hbm.at[slab, cols])

    return pl.kernel(
        body,
        out_type=jax.ShapeDtypeStruct((_NPAD, F), jnp.float32),
        mesh=_sc_mesh(),
        compiler_params=pltpu.CompilerParams(use_tc_tiling_on_sc=False),
        scratch_types=[
            pltpu.VMEM((_CPW, _CHUNK), jnp.int32),
            pltpu.VMEM((_CPW, _CHUNK), jnp.int32),
            [pltpu.VMEM((_CHUNK, HF), jnp.float32) for _ in range(_NBUF)],
            [pltpu.SemaphoreType.DMA for _ in range(_NBUF)],
            [pltpu.SemaphoreType.DMA for _ in range(_NBUF)],
            pltpu.VMEM_SHARED((_NPAD, HF), jnp.float32),
            pltpu.VMEM_SHARED((_NPAD, HF), jnp.float32),
        ],
    )


_CPWD = 80


def _make_deg():
    def body(ones_hbm, z_hbm, dst_hbm, out_hbm, dst_v, ones_v, sem, sem2,
             sem3, acc):
        cid = lax.axis_index("c")
        sid = lax.axis_index("s")
        wid = cid * _NS + sid
        slab = pl.ds(sid * _ROWS_PS, _ROWS_PS)
        pltpu.async_copy(z_hbm.at[slab], acc.at[slab], sem)
        pltpu.async_copy(dst_hbm.at[wid], dst_v, sem2)
        pltpu.async_copy(ones_hbm, ones_v, sem3)
        pltpu.make_async_copy(z_hbm.at[slab], acc.at[slab], sem).wait()
        pltpu.make_async_copy(dst_hbm.at[wid], dst_v, sem2).wait()
        pltpu.make_async_copy(ones_hbm, ones_v, sem3).wait()
        plsc.subcore_barrier()

        def fire(j, carry):
            pltpu.async_copy(ones_v, acc.at[dst_v.at[j]], sem, add=True)
            return carry

        lax.fori_loop(0, _CPWD, fire, 0)

        def drain(j, carry):
            pltpu.make_async_copy(ones_v, acc.at[dst_v.at[j]], sem).wait()
            return carry

        lax.fori_loop(0, _CPWD, drain, 0)
        plsc.subcore_barrier()
        pltpu.sync_copy(acc.at[slab], out_hbm.at[cid, slab])

    return pl.kernel(
        body,
        out_type=jax.ShapeDtypeStruct((_NC, _NPAD, 16), jnp.float32),
        mesh=_sc_mesh(),
        compiler_params=pltpu.CompilerParams(use_tc_tiling_on_sc=False),
        scratch_types=[
            pltpu.VMEM((_CPWD, _CHUNK), jnp.int32),
            pltpu.VMEM((_CHUNK, 16), jnp.float32),
            pltpu.SemaphoreType.DMA,
            pltpu.SemaphoreType.DMA,
            pltpu.SemaphoreType.DMA,
            pltpu.VMEM_SHARED((_NPAD, 16), jnp.float32),
        ],
    )


def _dinv_from(degp_ref):
    deg = degp_ref[0, :, 0:1] + degp_ref[1, :, 0:1] + 1.0
    return lax.rsqrt(deg)


def _first_body(x_ref, w_ref, degp_ref, g_ref):
    dinv = _dinv_from(degp_ref)
    g_ref[...] = dinv * jnp.dot(x_ref[...], w_ref[...],
                                preferred_element_type=jnp.float32)


def _mid2_body(sa_ref, sb_ref, degp_ref, b_ref, wa_ref, wb_ref, g_ref):
    dinv = _dinv_from(degp_ref)
    hw = b_ref.shape[-1] // 2
    aa = jnp.maximum(dinv * sa_ref[...] + b_ref[0:1, :hw], 0.0)
    ab = jnp.maximum(dinv * sb_ref[...] + b_ref[0:1, hw:], 0.0)
    g_ref[...] = dinv * (
        jnp.dot(aa, wa_ref[...], preferred_element_type=jnp.float32)
        + jnp.dot(ab, wb_ref[...], preferred_element_type=jnp.float32))


def _mid_body(s_ref, degp_ref, b_ref, w_ref, g_ref):
    dinv = _dinv_from(degp_ref)
    a = dinv * s_ref[...] + b_ref[0:1, :]
    a = jnp.maximum(a, 0.0)
    g_ref[...] = dinv * jnp.dot(a, w_ref[...], preferred_element_type=jnp.float32)


def _last_body(s_ref, degp_ref, b_ref, o_ref):
    dinv = _dinv_from(degp_ref)
    o_ref[...] = dinv * s_ref[...] + b_ref[0:1, :]


def _row_spec(F):
    return pl.BlockSpec((_BLK, F), lambda i: (i, 0))


def _partials_spec(F):
    return pl.BlockSpec((_NC, _BLK, F), lambda i: (0, i, 0))


def _full_spec(shape):
    nd = len(shape)
    return pl.BlockSpec(shape, lambda i: (0,) * nd)


def _tc_first(x_pad, W, degp):
    return pl.pallas_call(
        _first_body,
        grid=(_NPAD // _BLK,),
        in_specs=[_row_spec(128), _full_spec(W.shape), _partials_spec(16)],
        out_specs=_row_spec(W.shape[1]),
        out_shape=jax.ShapeDtypeStruct((_NPAD, W.shape[1]), jnp.float32),
    )(x_pad, W, degp)


def _tc_mid(s, degp, b8, W):
    F1, F2 = W.shape
    return pl.pallas_call(
        _mid_body,
        grid=(_NPAD // _BLK,),
        in_specs=[_row_spec(F1), _partials_spec(16), _full_spec(b8.shape),
                  _full_spec(W.shape)],
        out_specs=_row_spec(F2),
        out_shape=jax.ShapeDtypeStruct((_NPAD, F2), jnp.float32),
    )(s, degp, b8, W)


def _tc_mid2(sa, sb, degp, b8, Wa, Wb):
    F2 = Wa.shape[1]
    half = sa.shape[-1]
    return pl.pallas_call(
        _mid2_body,
        grid=(_NPAD // _BLK,),
        in_specs=[_row_spec(half), _row_spec(half), _partials_spec(16),
                  _full_spec(b8.shape), _full_spec(Wa.shape), _full_spec(Wb.shape)],
        out_specs=_row_spec(F2),
        out_shape=jax.ShapeDtypeStruct((_NPAD, F2), jnp.float32),
    )(sa, sb, degp, b8, Wa, Wb)


def _tc_last(s, degp, b8):
    F = s.shape[-1]
    return pl.pallas_call(
        _last_body,
        grid=(_NPAD // _BLK,),
        in_specs=[_row_spec(F), _partials_spec(16), _full_spec(b8.shape)],
        out_specs=_row_spec(F),
        out_shape=jax.ShapeDtypeStruct((_NPAD, F), jnp.float32),
    )(s, degp, b8)


def kernel(x, adj_t, W1, b1, W2, b2, W3, b3):
    src = adj_t[0].astype(jnp.int32)
    dst = adj_t[1].astype(jnp.int32)
    pad = _EPAD - _E
    src_flat = jnp.concatenate([src, jnp.zeros((pad,), jnp.int32)])
    dst_flat = jnp.concatenate([dst, jnp.full((pad,), _N, jnp.int32)])
    srcp = src_flat.reshape(_NS, _CPW, _CHUNK)
    dstp = dst_flat.reshape(_NS, _CPW, _CHUNK)
    dstp_deg = dst_flat.reshape(_NC * _NS, _CPWD, _CHUNK)
    x_pad = jnp.pad(x, ((0, _NPAD - _N), (0, 0)))
    ones16 = jnp.ones((_CHUNK, 16), jnp.float32)

    degp = _make_deg()(ones16, jnp.zeros((_NPAD, 16), jnp.float32), dstp_deg)

    g1 = _tc_first(x_pad, W1, degp)
    scat64 = _make_scatter(64)
    s1a = scat64(g1[:, :64], srcp, dstp)
    s1b = scat64(g1[:, 64:], srcp, dstp)

    b1_8 = jnp.tile(b1[None, :], (8, 1))
    g2 = _tc_mid2(s1a, s1b, degp, b1_8, W2[:64], W2[64:])
    s2 = scat64(g2, srcp, dstp)

    W3p = jnp.pad(W3, ((0, 0), (0, 24)))
    b2_8 = jnp.tile(b2[None, :], (8, 1))
    g3 = _tc_mid(s2, degp, b2_8, W3p)
    s3 = scat64(g3, srcp, dstp)

    b3_8 = jnp.tile(jnp.pad(b3, (0, 24))[None, :], (8, 1))
    out = _tc_last(s3, degp, b3_8)
    return out[:_N, :40]


# speedup vs baseline: 1.0958x; 1.0117x over previous
import functools

import jax
import jax.numpy as jnp
from jax import lax
from jax.experimental import pallas as pl
from jax.experimental.pallas import tpu as pltpu
from jax.experimental.pallas import tpu_sc as plsc

_N = 10000
_NPAD = 10240
_E = 320000
_NC = 2
_NS = 16
_CHUNK = 128
_CPW = 160
_EPW = _CPW * _CHUNK
_EPAD = _EPW * _NS
_ROWS_PS = _NPAD // _NS
_NBUF = 8
_BLK = 2560


def _sc_mesh():
    return plsc.VectorSubcoreMesh(core_axis_name="c", subcore_axis_name="s")


@functools.lru_cache(maxsize=None)
def _make_scatter(F):
    HF = F // 2

    def body(g_hbm, src_hbm, dst_hbm, out_hbm,
             src_v, dst_v, bufs, semg, sems, acc, gsp):
        cid = lax.axis_index("c")
        sid = lax.axis_index("s")
        slab = pl.ds(sid * _ROWS_PS, _ROWS_PS)
        cols = pl.ds(cid * HF, HF)

        pltpu.async_copy(g_hbm.at[slab, cols], gsp.at[slab], semg[0])
        pltpu.async_copy(g_hbm.at[slab, cols], acc.at[slab], semg[1])
        pltpu.async_copy(src_hbm.at[sid], src_v, semg[2])
        pltpu.async_copy(dst_hbm.at[sid], dst_v, semg[3])
        pltpu.make_async_copy(g_hbm.at[slab, cols], gsp.at[slab], semg[0]).wait()
        pltpu.make_async_copy(g_hbm.at[slab, cols], acc.at[slab], semg[1]).wait()
        pltpu.make_async_copy(src_hbm.at[sid], src_v, semg[2]).wait()
        pltpu.make_async_copy(dst_hbm.at[sid], dst_v, semg[3]).wait()
        plsc.subcore_barrier()

        for b in range(_NBUF):
            pltpu.async_copy(gsp.at[src_v.at[b]], bufs[b], semg[b])

        def step(p, carry):
            base = p * _NBUF
            for b in range(_NBUF):
                j = base + b
                pltpu.make_async_copy(gsp.at[src_v.at[j]], bufs[b],
                                      semg[b]).wait()
                pltpu.async_copy(bufs[b], acc.at[dst_v.at[j]], sems[b],
                                 add=True)
            for b in range(_NBUF):
                j2 = base + _NBUF + b

                @pl.when(j2 < _CPW)
                def _():
                    pltpu.make_async_copy(bufs[b], acc.at[dst_v.at[base + b]],
                                          sems[b]).wait()
                    pltpu.async_copy(gsp.at[src_v.at[j2]], bufs[b], semg[b])

            return carry

        lax.fori_loop(0, _CPW // _NBUF, step, 0)
        for b in range(_NBUF):
            j = _CPW - _NBUF + b
            pltpu.make_async_copy(bufs[b], acc.at[dst_v.at[j]], sems[b]).wait()
        plsc.subcore_barrier()
        pltpu.sync_copy(acc.at[slab], out_hbm.at[slab, cols])

    return pl.kernel(
        body,
        out_type=jax.ShapeDtypeStruct((_NPAD, F), jnp.float32),
        mesh=_sc_mesh(),
        compiler_params=pltpu.CompilerParams(use_tc_tiling_on_sc=False),
        scratch_types=[
            pltpu.VMEM((_CPW, _CHUNK), jnp.int32),
            pltpu.VMEM((_CPW, _CHUNK), jnp.int32),
            [pltpu.VMEM((_CHUNK, HF), jnp.float32) for _ in range(_NBUF)],
            [pltpu.SemaphoreType.DMA for _ in range(_NBUF)],
            [pltpu.SemaphoreType.DMA for _ in range(_NBUF)],
            pltpu.VMEM_SHARED((_NPAD, HF), jnp.float32),
            pltpu.VMEM_SHARED((_NPAD, HF), jnp.float32),
        ],
    )


_CPWD = 80


def _make_deg():
    def body(ones_hbm, z_hbm, dst_hbm, out_hbm, dst_v, ones_v, sem, sem2,
             sem3, acc):
        cid = lax.axis_index("c")
        sid = lax.axis_index("s")
        wid = cid * _NS + sid
        slab = pl.ds(sid * _ROWS_PS, _ROWS_PS)
        pltpu.async_copy(z_hbm.at[slab], acc.at[slab], sem)
        pltpu.async_copy(dst_hbm.at[wid], dst_v, sem2)
        pltpu.async_copy(ones_hbm, ones_v, sem3)
        pltpu.make_async_copy(z_hbm.at[slab], acc.at[slab], sem).wait()
        pltpu.make_async_copy(dst_hbm.at[wid], dst_v, sem2).wait()
        pltpu.make_async_copy(ones_hbm, ones_v, sem3).wait()
        plsc.subcore_barrier()

        def fire(j, carry):
            pltpu.async_copy(ones_v, acc.at[dst_v.at[j]], sem, add=True)
            return carry

        lax.fori_loop(0, _CPWD, fire, 0)

        def drain(j, carry):
            pltpu.make_async_copy(ones_v, acc.at[dst_v.at[j]], sem).wait()
            return carry

        lax.fori_loop(0, _CPWD, drain, 0)
        plsc.subcore_barrier()
        pltpu.sync_copy(acc.at[slab], out_hbm.at[cid, slab])

    return pl.kernel(
        body,
        out_type=jax.ShapeDtypeStruct((_NC, _NPAD, 16), jnp.float32),
        mesh=_sc_mesh(),
        compiler_params=pltpu.CompilerParams(use_tc_tiling_on_sc=False),
        scratch_types=[
            pltpu.VMEM((_CPWD, _CHUNK), jnp.int32),
            pltpu.VMEM((_CHUNK, 16), jnp.float32),
            pltpu.SemaphoreType.DMA,
            pltpu.SemaphoreType.DMA,
            pltpu.SemaphoreType.DMA,
            pltpu.VMEM_SHARED((_NPAD, 16), jnp.float32),
        ],
    )


def _dinv_from(degp_ref):
    deg = degp_ref[0, :, 0:1] + degp_ref[1, :, 0:1] + 1.0
    return lax.rsqrt(deg)


def _first_body(x_ref, w_ref, degp_ref, g_ref):
    dinv = _dinv_from(degp_ref)
    g_ref[...] = dinv * jnp.dot(x_ref[...], w_ref[...],
                                preferred_element_type=jnp.float32)


def _mid2_body(sa_ref, sb_ref, degp_ref, b_ref, wa_ref, wb_ref, g_ref):
    dinv = _dinv_from(degp_ref)
    hw = b_ref.shape[-1] // 2
    aa = jnp.maximum(dinv * sa_ref[...] + b_ref[0:1, :hw], 0.0)
    ab = jnp.maximum(dinv * sb_ref[...] + b_ref[0:1, hw:], 0.0)
    g_ref[...] = dinv * (
        jnp.dot(aa, wa_ref[...], preferred_element_type=jnp.float32)
        + jnp.dot(ab, wb_ref[...], preferred_element_type=jnp.float32))


def _mid_body(s_ref, degp_ref, b_ref, w_ref, g_ref):
    dinv = _dinv_from(degp_ref)
    a = dinv * s_ref[...] + b_ref[0:1, :]
    a = jnp.maximum(a, 0.0)
    g_ref[...] = dinv * jnp.dot(a, w_ref[...], preferred_element_type=jnp.float32)


def _last_body(s_ref, degp_ref, b_ref, o_ref):
    dinv = _dinv_from(degp_ref)
    o_ref[...] = dinv * s_ref[...] + b_ref[0:1, :]


def _row_spec(F):
    return pl.BlockSpec((_BLK, F), lambda i: (i, 0))


def _partials_spec(F):
    return pl.BlockSpec((_NC, _BLK, F), lambda i: (0, i, 0))


def _full_spec(shape):
    nd = len(shape)
    return pl.BlockSpec(shape, lambda i: (0,) * nd)


def _tc_first(x_pad, W, degp):
    return pl.pallas_call(
        _first_body,
        grid=(_NPAD // _BLK,),
        in_specs=[_row_spec(128), _full_spec(W.shape), _partials_spec(16)],
        out_specs=_row_spec(W.shape[1]),
        out_shape=jax.ShapeDtypeStruct((_NPAD, W.shape[1]), jnp.float32),
    )(x_pad, W, degp)


def _tc_mid(s, degp, b8, W):
    F1, F2 = W.shape
    return pl.pallas_call(
        _mid_body,
        grid=(_NPAD // _BLK,),
        in_specs=[_row_spec(F1), _partials_spec(16), _full_spec(b8.shape),
                  _full_spec(W.shape)],
        out_specs=_row_spec(F2),
        out_shape=jax.ShapeDtypeStruct((_NPAD, F2), jnp.float32),
    )(s, degp, b8, W)


def _tc_mid2(sa, sb, degp, b8, Wa, Wb):
    F2 = Wa.shape[1]
    half = sa.shape[-1]
    return pl.pallas_call(
        _mid2_body,
        grid=(_NPAD // _BLK,),
        in_specs=[_row_spec(half), _row_spec(half), _partials_spec(16),
                  _full_spec(b8.shape), _full_spec(Wa.shape), _full_spec(Wb.shape)],
        out_specs=_row_spec(F2),
        out_shape=jax.ShapeDtypeStruct((_NPAD, F2), jnp.float32),
    )(sa, sb, degp, b8, Wa, Wb)


def _tc_last(s, degp, b8):
    F = s.shape[-1]
    return pl.pallas_call(
        _last_body,
        grid=(_NPAD // _BLK,),
        in_specs=[_row_spec(F), _partials_spec(16), _full_spec(b8.shape)],
        out_specs=_row_spec(F),
        out_shape=jax.ShapeDtypeStruct((_NPAD, F), jnp.float32),
    )(s, degp, b8)


def kernel(x, adj_t, W1, b1, W2, b2, W3, b3):
    src = adj_t[0].astype(jnp.int32)
    dst = adj_t[1].astype(jnp.int32)
    pad = _EPAD - _E
    src_flat = jnp.concatenate([src, jnp.zeros((pad,), jnp.int32)])
    dst_flat = jnp.concatenate([dst, jnp.full((pad,), _N, jnp.int32)])
    srcp = src_flat.reshape(_NS, _CPW, _CHUNK)
    dstp = dst_flat.reshape(_NS, _CPW, _CHUNK)
    dstp_deg = dst_flat.reshape(_NC * _NS, _CPWD, _CHUNK)
    x_pad = jnp.pad(x, ((0, _NPAD - _N), (0, 0)))
    ones16 = jnp.ones((_CHUNK, 16), jnp.float32)

    degp = _make_deg()(ones16, jnp.zeros((_NPAD, 16), jnp.float32), dstp_deg)

    g1 = _tc_first(x_pad, W1, degp)
    scat64 = _make_scatter(64)
    s1a = scat64(g1[:, :64], srcp, dstp)
    s1b = scat64(g1[:, 64:], srcp, dstp)

    b1_8 = jnp.tile(b1[None, :], (8, 1))
    g2 = _tc_mid2(s1a, s1b, degp, b1_8, W2[:64], W2[64:])
    s2 = scat64(g2, srcp, dstp)

    W3p = jnp.pad(W3, ((0, 0), (0, 24)))
    b2_8 = jnp.tile(b2[None, :], (8, 1))
    g3 = _tc_mid(s2, degp, b2_8, W3p)
    s3 = scat64(g3, srcp, dstp)

    b3_8 = jnp.tile(jnp.pad(b3, (0, 24))[None, :], (8, 1))
    out = _tc_last(s3, degp, b3_8)
    return out[:_N, :40]


# speedup vs baseline: 1.1001x; 1.0039x over previous
import functools

import jax
import jax.numpy as jnp
from jax import lax
from jax.experimental import pallas as pl
from jax.experimental.pallas import tpu as pltpu
from jax.experimental.pallas import tpu_sc as plsc

_N = 10000
_NPAD = 10240
_E = 320000
_NC = 2
_NS = 16
_CHUNK = 128
_CPW = 160
_EPW = _CPW * _CHUNK
_EPAD = _EPW * _NS
_ROWS_PS = _NPAD // _NS
_NBUF = 8
_BLK = 5120


def _sc_mesh():
    return plsc.VectorSubcoreMesh(core_axis_name="c", subcore_axis_name="s")


@functools.lru_cache(maxsize=None)
def _make_scatter(F):
    HF = F // 2

    def body(g_hbm, src_hbm, dst_hbm, out_hbm,
             src_v, dst_v, bufs, semg, sems, acc, gsp):
        cid = lax.axis_index("c")
        sid = lax.axis_index("s")
        slab = pl.ds(sid * _ROWS_PS, _ROWS_PS)
        cols = pl.ds(cid * HF, HF)

        pltpu.async_copy(g_hbm.at[slab, cols], gsp.at[slab], semg[0])
        pltpu.async_copy(g_hbm.at[slab, cols], acc.at[slab], semg[1])
        pltpu.async_copy(src_hbm.at[sid], src_v, semg[2])
        pltpu.async_copy(dst_hbm.at[sid], dst_v, semg[3])
        pltpu.make_async_copy(g_hbm.at[slab, cols], gsp.at[slab], semg[0]).wait()
        pltpu.make_async_copy(g_hbm.at[slab, cols], acc.at[slab], semg[1]).wait()
        pltpu.make_async_copy(src_hbm.at[sid], src_v, semg[2]).wait()
        pltpu.make_async_copy(dst_hbm.at[sid], dst_v, semg[3]).wait()
        plsc.subcore_barrier()

        for b in range(_NBUF):
            pltpu.async_copy(gsp.at[src_v.at[b]], bufs[b], semg[b])

        def step(p, carry):
            base = p * _NBUF
            for b in range(_NBUF):
                j = base + b
                pltpu.make_async_copy(gsp.at[src_v.at[j]], bufs[b],
                                      semg[b]).wait()
                pltpu.async_copy(bufs[b], acc.at[dst_v.at[j]], sems[b],
                                 add=True)
            for b in range(_NBUF):
                j2 = base + _NBUF + b

                @pl.when(j2 < _CPW)
                def _():
                    pltpu.make_async_copy(bufs[b], acc.at[dst_v.at[base + b]],
                                          sems[b]).wait()
                    pltpu.async_copy(gsp.at[src_v.at[j2]], bufs[b], semg[b])

            return carry

        lax.fori_loop(0, _CPW // _NBUF, step, 0)
        for b in range(_NBUF):
            j = _CPW - _NBUF + b
            pltpu.make_async_copy(bufs[b], acc.at[dst_v.at[j]], sems[b]).wait()
        plsc.subcore_barrier()
        pltpu.sync_copy(acc.at[slab], out_hbm.at[slab, cols])

    return pl.kernel(
        body,
        out_type=jax.ShapeDtypeStruct((_NPAD, F), jnp.float32),
        mesh=_sc_mesh(),
        compiler_params=pltpu.CompilerParams(use_tc_tiling_on_sc=False),
        scratch_types=[
            pltpu.VMEM((_CPW, _CHUNK), jnp.int32),
            pltpu.VMEM((_CPW, _CHUNK), jnp.int32),
            [pltpu.VMEM((_CHUNK, HF), jnp.float32) for _ in range(_NBUF)],
            [pltpu.SemaphoreType.DMA for _ in range(_NBUF)],
            [pltpu.SemaphoreType.DMA for _ in range(_NBUF)],
            pltpu.VMEM_SHARED((_NPAD, HF), jnp.float32),
            pltpu.VMEM_SHARED((_NPAD, HF), jnp.float32),
        ],
    )


_CPWD = 80


def _make_deg():
    def body(ones_hbm, z_hbm, dst_hbm, out_hbm, dst_v, ones_v, sem, sem2,
             sem3, acc):
        cid = lax.axis_index("c")
        sid = lax.axis_index("s")
        wid = cid * _NS + sid
        slab = pl.ds(sid * _ROWS_PS, _ROWS_PS)
        pltpu.async_copy(z_hbm.at[slab], acc.at[slab], sem)
        pltpu.async_copy(dst_hbm.at[wid], dst_v, sem2)
        pltpu.async_copy(ones_hbm, ones_v, sem3)
        pltpu.make_async_copy(z_hbm.at[slab], acc.at[slab], sem).wait()
        pltpu.make_async_copy(dst_hbm.at[wid], dst_v, sem2).wait()
        pltpu.make_async_copy(ones_hbm, ones_v, sem3).wait()
        plsc.subcore_barrier()

        def fire(j, carry):
            pltpu.async_copy(ones_v, acc.at[dst_v.at[j]], sem, add=True)
            return carry

        lax.fori_loop(0, _CPWD, fire, 0)

        def drain(j, carry):
            pltpu.make_async_copy(ones_v, acc.at[dst_v.at[j]], sem).wait()
            return carry

        lax.fori_loop(0, _CPWD, drain, 0)
        plsc.subcore_barrier()
        pltpu.sync_copy(acc.at[slab], out_hbm.at[cid, slab])

    return pl.kernel(
        body,
        out_type=jax.ShapeDtypeStruct((_NC, _NPAD, 16), jnp.float32),
        mesh=_sc_mesh(),
        compiler_params=pltpu.CompilerParams(use_tc_tiling_on_sc=False),
        scratch_types=[
            pltpu.VMEM((_CPWD, _CHUNK), jnp.int32),
            pltpu.VMEM((_CHUNK, 16), jnp.float32),
            pltpu.SemaphoreType.DMA,
            pltpu.SemaphoreType.DMA,
            pltpu.SemaphoreType.DMA,
            pltpu.VMEM_SHARED((_NPAD, 16), jnp.float32),
        ],
    )


def _dinv_from(degp_ref):
    deg = degp_ref[0, :, 0:1] + degp_ref[1, :, 0:1] + 1.0
    return lax.rsqrt(deg)


def _first_body(x_ref, w_ref, degp_ref, g_ref):
    dinv = _dinv_from(degp_ref)
    g_ref[...] = dinv * jnp.dot(x_ref[...], w_ref[...],
                                preferred_element_type=jnp.float32)


def _mid2_body(sa_ref, sb_ref, degp_ref, b_ref, wa_ref, wb_ref, g_ref):
    dinv = _dinv_from(degp_ref)
    hw = b_ref.shape[-1] // 2
    aa = jnp.maximum(dinv * sa_ref[...] + b_ref[0:1, :hw], 0.0)
    ab = jnp.maximum(dinv * sb_ref[...] + b_ref[0:1, hw:], 0.0)
    g_ref[...] = dinv * (
        jnp.dot(aa, wa_ref[...], preferred_element_type=jnp.float32)
        + jnp.dot(ab, wb_ref[...], preferred_element_type=jnp.float32))


def _mid_body(s_ref, degp_ref, b_ref, w_ref, g_ref):
    dinv = _dinv_from(degp_ref)
    a = dinv * s_ref[...] + b_ref[0:1, :]
    a = jnp.maximum(a, 0.0)
    g_ref[...] = dinv * jnp.dot(a, w_ref[...], preferred_element_type=jnp.float32)


def _last_body(s_ref, degp_ref, b_ref, o_ref):
    dinv = _dinv_from(degp_ref)
    o_ref[...] = dinv * s_ref[...] + b_ref[0:1, :]


def _row_spec(F):
    return pl.BlockSpec((_BLK, F), lambda i: (i, 0))


def _partials_spec(F):
    return pl.BlockSpec((_NC, _BLK, F), lambda i: (0, i, 0))


def _full_spec(shape):
    nd = len(shape)
    return pl.BlockSpec(shape, lambda i: (0,) * nd)


def _tc_first(x_pad, W, degp):
    return pl.pallas_call(
        _first_body,
        grid=(_NPAD // _BLK,),
        in_specs=[_row_spec(128), _full_spec(W.shape), _partials_spec(16)],
        out_specs=_row_spec(W.shape[1]),
        out_shape=jax.ShapeDtypeStruct((_NPAD, W.shape[1]), jnp.float32),
    )(x_pad, W, degp)


def _tc_mid(s, degp, b8, W):
    F1, F2 = W.shape
    return pl.pallas_call(
        _mid_body,
        grid=(_NPAD // _BLK,),
        in_specs=[_row_spec(F1), _partials_spec(16), _full_spec(b8.shape),
                  _full_spec(W.shape)],
        out_specs=_row_spec(F2),
        out_shape=jax.ShapeDtypeStruct((_NPAD, F2), jnp.float32),
    )(s, degp, b8, W)


def _tc_mid2(sa, sb, degp, b8, Wa, Wb):
    F2 = Wa.shape[1]
    half = sa.shape[-1]
    return pl.pallas_call(
        _mid2_body,
        grid=(_NPAD // _BLK,),
        in_specs=[_row_spec(half), _row_spec(half), _partials_spec(16),
                  _full_spec(b8.shape), _full_spec(Wa.shape), _full_spec(Wb.shape)],
        out_specs=_row_spec(F2),
        out_shape=jax.ShapeDtypeStruct((_NPAD, F2), jnp.float32),
    )(sa, sb, degp, b8, Wa, Wb)


def _tc_last(s, degp, b8):
    F = s.shape[-1]
    return pl.pallas_call(
        _last_body,
        grid=(_NPAD // _BLK,),
        in_specs=[_row_spec(F), _partials_spec(16), _full_spec(b8.shape)],
        out_specs=_row_spec(F),
        out_shape=jax.ShapeDtypeStruct((_NPAD, F), jnp.float32),
    )(s, degp, b8)


def kernel(x, adj_t, W1, b1, W2, b2, W3, b3):
    src = adj_t[0].astype(jnp.int32)
    dst = adj_t[1].astype(jnp.int32)
    pad = _EPAD - _E
    src_flat = jnp.concatenate([src, jnp.zeros((pad,), jnp.int32)])
    dst_flat = jnp.concatenate([dst, jnp.full((pad,), _N, jnp.int32)])
    srcp = src_flat.reshape(_NS, _CPW, _CHUNK)
    dstp = dst_flat.reshape(_NS, _CPW, _CHUNK)
    dstp_deg = dst_flat.reshape(_NC * _NS, _CPWD, _CHUNK)
    x_pad = jnp.pad(x, ((0, _NPAD - _N), (0, 0)))
    ones16 = jnp.ones((_CHUNK, 16), jnp.float32)

    degp = _make_deg()(ones16, jnp.zeros((_NPAD, 16), jnp.float32), dstp_deg)

    g1 = _tc_first(x_pad, W1, degp)
    scat64 = _make_scatter(64)
    s1a = scat64(g1[:, :64], srcp, dstp)
    s1b = scat64(g1[:, 64:], srcp, dstp)

    b1_8 = jnp.tile(b1[None, :], (8, 1))
    g2 = _tc_mid2(s1a, s1b, degp, b1_8, W2[:64], W2[64:])
    s2 = scat64(g2, srcp, dstp)

    W3p = jnp.pad(W3, ((0, 0), (0, 24)))
    b2_8 = jnp.tile(b2[None, :], (8, 1))
    g3 = _tc_mid(s2, degp, b2_8, W3p)
    s3 = scat64(g3, srcp, dstp)

    b3_8 = jnp.tile(jnp.pad(b3, (0, 24))[None, :], (8, 1))
    out = _tc_last(s3, degp, b3_8)
    return out[:_N, :40]
